# baseline (speedup 1.0000x reference)
import jax
import jax.numpy as jnp
from jax import lax
from jax.experimental import pallas as pl
from jax.experimental.pallas import tpu as pltpu
from jax.experimental.pallas import tpu_sc as plsc

N = 10000
NP = 10240
E = 320000
EP = 327680
G = 64
F = 128
NC, NS = 2, 16
NW = NC * NS
K = 64
NB = 4
EPT = EP // NW
NCH = EPT // K
RPT = NP // NS
SRC_PAD = 10232
DST_PAD = 10239

_MESH = plsc.VectorSubcoreMesh(core_axis_name="c", subcore_axis_name="s")


def _deg_body(dst_hbm, out_hbm, idx_v, hist_v):
    c = lax.axis_index("c")
    s = lax.axis_index("s")
    w = c * NS + s
    pltpu.sync_copy(dst_hbm.at[w], idx_v)

    def zero(i, _):
        hist_v[pl.ds(i * 16, 16)] = jnp.zeros((16,), jnp.float32)
        return 0
    lax.fori_loop(0, NP // 16, zero, 0)

    def add(i, _):
        idx = idx_v[i, :]
        plsc.addupdate_scatter(hist_v, [idx], jnp.ones((16,), jnp.float32))
        return 0
    lax.fori_loop(0, EPT // 16, add, 0)
    pltpu.sync_copy(hist_v, out_hbm.at[w, 0])


_deg_call = pl.kernel(
    _deg_body, mesh=_MESH,
    out_type=jax.ShapeDtypeStruct((NW, 1, NP), jnp.float32),
    scratch_types=[
        pltpu.VMEM((EPT // 16, 16), jnp.int32),
        pltpu.VMEM((NP,), jnp.float32),
    ],
    compiler_params=pltpu.CompilerParams(needs_layout_passes=False),
)


Q0 = 160
Q1 = 160
QMAX = max(Q0, Q1)
R0 = Q0 // NB
R1 = Q1 // NB
RMAX = max(R0, R1)


def _scatter_body(g_hbm, src0_hbm, dst0_hbm, src1_hbm, dst1_hbm, out_hbm,
                  isrc_v, idst_v, rows_v, acc_sh, isem, gsem, ssem):
    c = lax.axis_index("c")
    s = lax.axis_index("s")
    myrounds = jnp.where(c == 0, R0, R1)

    @pl.when(c == 0)
    def _():
        pltpu.sync_copy(src0_hbm.at[s, pl.ds(0, NB)], isrc_v.at[0])
        pltpu.sync_copy(dst0_hbm.at[s, pl.ds(0, NB)], idst_v.at[0])

    @pl.when(c == 1)
    def _():
        pltpu.sync_copy(src1_hbm.at[s, pl.ds(0, NB)], isrc_v.at[0])
        pltpu.sync_copy(dst1_hbm.at[s, pl.ds(0, NB)], idst_v.at[0])

    def zrow(i, _):
        for jj in range(F // 16):
            rows_v[0, i, pl.ds(jj * 16, 16)] = jnp.zeros((16,), jnp.float32)
        return 0
    lax.fori_loop(0, K, zrow, 0, unroll=4)

    def cp(i, _):
        pltpu.sync_copy(rows_v.at[0], acc_sh.at[pl.ds(s * RPT + i * K, K)])
        return 0
    lax.fori_loop(0, RPT // K, cp, 0)
    plsc.subcore_barrier()

    def rnd(r, _):
        slot = lax.rem(r, 2)
        nslot = 1 - slot

        @pl.when(r + 1 < myrounds)
        def _():
            base = (r + 1) * NB

            @pl.when(c == 0)
            def _():
                pltpu.async_copy(src0_hbm.at[s, pl.ds(base, NB)],
                                 isrc_v.at[nslot], isem)
                pltpu.async_copy(dst0_hbm.at[s, pl.ds(base, NB)],
                                 idst_v.at[nslot], isem)

            @pl.when(c == 1)
            def _():
                pltpu.async_copy(src1_hbm.at[s, pl.ds(base, NB)],
                                 isrc_v.at[nslot], isem)
                pltpu.async_copy(dst1_hbm.at[s, pl.ds(base, NB)],
                                 idst_v.at[nslot], isem)

        @pl.when(r < myrounds)
        def _():
            gd = []
            for b in range(NB):
                gd.append(pltpu.async_copy(
                    g_hbm.at[isrc_v.at[slot, b, 0]], rows_v.at[b], gsem))
            sd = []
            for b in range(NB):
                gd[b].wait()
                sd.append(pltpu.async_copy(
                    rows_v.at[b], acc_sh.at[idst_v.at[slot, b, 0]],
                    ssem, add=True))
            for d in sd:
                d.wait()

        @pl.when(r + 1 < myrounds)
        def _():
            pltpu.make_async_copy(src0_hbm.at[s, pl.ds(0, NB)],
                                  isrc_v.at[nslot], isem).wait()
            pltpu.make_async_copy(dst0_hbm.at[s, pl.ds(0, NB)],
                                  idst_v.at[nslot], isem).wait()
        return 0
    lax.fori_loop(0, RMAX, rnd, 0)
    plsc.subcore_barrier()
    pltpu.sync_copy(acc_sh.at[pl.ds(s * RPT, RPT)],
                    out_hbm.at[c, pl.ds(s * RPT, RPT)])


_scatter_call = pl.kernel(
    _scatter_body, mesh=_MESH,
    out_type=jax.ShapeDtypeStruct((NC, NP, F), jnp.float32),
    scratch_types=[
        pltpu.VMEM((2, NB, 1, K), jnp.int32),
        pltpu.VMEM((2, NB, 1, K), jnp.int32),
        pltpu.VMEM((NB, K, F), jnp.float32),
        pltpu.VMEM_SHARED((NP, F), jnp.float32),
        pltpu.SemaphoreType.DMA,
        pltpu.SemaphoreType.DMA,
        pltpu.SemaphoreType.DMA,
    ],
)


BLK = 1024
GRID = NP // BLK


def _prep_body(x_ref, degp_ref, eye_ref, batch_ref, xcat_ref, w1xt_ref,
               w1ct_ref, g1_ref, dinv_ref):
    deg_row = jnp.sum(degp_ref[...], axis=0) + 1.0
    deg = lax.dot_general(eye_ref[...], deg_row, (((1,), (1,)), ((), ())),
                          preferred_element_type=jnp.float32)
    dinv = lax.rsqrt(deg)
    onehot = (batch_ref[...] ==
              lax.broadcasted_iota(jnp.int32, (BLK, G), 1)).astype(jnp.float32)
    xcw = jnp.dot(xcat_ref[...], w1ct_ref[...],
                  preferred_element_type=jnp.float32)
    hw = (jnp.dot(x_ref[...], w1xt_ref[...],
                  preferred_element_type=jnp.float32)
          + jnp.dot(onehot, xcw, preferred_element_type=jnp.float32))
    g1_ref[...] = dinv * hw
    dinv_ref[...] = dinv


def _prep(x, degp, eye, batch2d, xcat, w1xt, w1ct):
    return pl.pallas_call(
        _prep_body,
        grid=(GRID,),
        in_specs=[
            pl.BlockSpec((BLK, F), lambda i: (i, 0)),
            pl.BlockSpec((NW, 1, BLK), lambda i: (0, 0, i)),
            pl.BlockSpec((BLK, BLK), lambda i: (0, 0)),
            pl.BlockSpec((BLK, 1), lambda i: (i, 0)),
            pl.BlockSpec((G, 8), lambda i: (0, 0)),
            pl.BlockSpec((F, F), lambda i: (0, 0)),
            pl.BlockSpec((8, F), lambda i: (0, 0)),
        ],
        out_specs=[
            pl.BlockSpec((BLK, F), lambda i: (i, 0)),
            pl.BlockSpec((BLK, 1), lambda i: (i, 0)),
        ],
        out_shape=[
            jax.ShapeDtypeStruct((NP, F), jnp.float32),
            jax.ShapeDtypeStruct((NP, 1), jnp.float32),
        ],
    )(x, degp, eye, batch2d, xcat, w1xt, w1ct)


def _comb1_body(p_ref, g1_ref, dinv_ref, b1_ref, w2t_ref, g2_ref):
    dinv = dinv_ref[...]
    h1 = jnp.maximum(
        dinv * (p_ref[0] + p_ref[1] + g1_ref[...]) + b1_ref[...], 0.0)
    g2_ref[...] = dinv * jnp.dot(h1, w2t_ref[...],
                                 preferred_element_type=jnp.float32)


def _comb1(p, g1, dinv, b1, w2t):
    return pl.pallas_call(
        _comb1_body,
        grid=(GRID,),
        in_specs=[
            pl.BlockSpec((NC, BLK, F), lambda i: (0, i, 0)),
            pl.BlockSpec((BLK, F), lambda i: (i, 0)),
            pl.BlockSpec((BLK, 1), lambda i: (i, 0)),
            pl.BlockSpec((1, F), lambda i: (0, 0)),
            pl.BlockSpec((F, F), lambda i: (0, 0)),
        ],
        out_specs=pl.BlockSpec((BLK, F), lambda i: (i, 0)),
        out_shape=jax.ShapeDtypeStruct((NP, F), jnp.float32),
    )(p, g1, dinv, b1, w2t)


def _comb2_body(q_ref, g2_ref, dinv_ref, b2_ref, batch_ref, xs_ref, cnt_ref):
    i = pl.program_id(0)
    dinv = dinv_ref[...]
    h2 = jnp.maximum(
        dinv * (q_ref[0] + q_ref[1] + g2_ref[...]) + b2_ref[...], 0.0)
    onehot = (batch_ref[...] ==
              lax.broadcasted_iota(jnp.int32, (BLK, G), 1)).astype(jnp.float32)
    xs_p = lax.dot_general(onehot, h2, (((0,), (0,)), ((), ())),
                           preferred_element_type=jnp.float32)
    cnt_p = jnp.broadcast_to(jnp.sum(onehot, axis=0)[:, None], (G, F))

    @pl.when(i == 0)
    def _():
        xs_ref[...] = xs_p
        cnt_ref[...] = cnt_p

    @pl.when(i > 0)
    def _():
        xs_ref[...] += xs_p
        cnt_ref[...] += cnt_p


def _comb2(q, g2, dinv, b2, batch2d):
    return pl.pallas_call(
        _comb2_body,
        grid=(GRID,),
        in_specs=[
            pl.BlockSpec((NC, BLK, F), lambda i: (0, i, 0)),
            pl.BlockSpec((BLK, F), lambda i: (i, 0)),
            pl.BlockSpec((BLK, 1), lambda i: (i, 0)),
            pl.BlockSpec((1, F), lambda i: (0, 0)),
            pl.BlockSpec((BLK, 1), lambda i: (i, 0)),
        ],
        out_specs=[
            pl.BlockSpec((G, F), lambda i: (0, 0)),
            pl.BlockSpec((G, F), lambda i: (0, 0)),
        ],
        out_shape=[
            jax.ShapeDtypeStruct((G, F), jnp.float32),
            jax.ShapeDtypeStruct((G, F), jnp.float32),
        ],
        compiler_params=pltpu.CompilerParams(
            dimension_semantics=("arbitrary",)),
    )(q, g2, dinv, b2, batch2d)


def _head_body(xin_ref, xs_ref, cnt_ref, wnt_ref, bn_ref, wlt_ref, bl_ref,
               out_ref):
    xs = xs_ref[...] / jnp.maximum(cnt_ref[...], 1.0)
    z = jnp.maximum(jnp.dot(xs, wnt_ref[...],
                            preferred_element_type=jnp.float32)
                    + bn_ref[...], 0.0)
    out = jnp.dot(z, wlt_ref[...], preferred_element_type=jnp.float32) \
        + bl_ref[...]
    dev = xin_ref[:, 0:1]
    out_ref[...] = dev * (1.0 + out[:, 1:2]) - out[:, 0:1]


def _head(x_in, xs, cnt, wnt, bn, wlt, bl):
    return pl.pallas_call(
        _head_body,
        out_shape=jax.ShapeDtypeStruct((G, 1), jnp.float32),
    )(x_in, xs, cnt, wnt, bn, wlt, bl)


def kernel(x_in, x, edge_index, batch, conv1_weight, conv1_bias,
           conv2_weight, conv2_bias, nnl_weight, nnl_bias,
           linX_weight, linX_bias):
    src_flat = jnp.concatenate([
        edge_index[0].astype(jnp.int32),
        jnp.full((EP - E,), SRC_PAD, jnp.int32)])
    dst_flat = jnp.concatenate([
        edge_index[1].astype(jnp.int32),
        jnp.full((EP - E,), DST_PAD, jnp.int32)])
    n0 = NS * Q0 * K
    src0 = src_flat[:n0].reshape(NS, Q0, 1, K)
    dst0 = dst_flat[:n0].reshape(NS, Q0, 1, K)
    src1 = src_flat[n0:].reshape(NS, Q1, 1, K)
    dst1 = dst_flat[n0:].reshape(NS, Q1, 1, K)
    dst_d = dst_flat.reshape(NW, EPT // 16, 16)
    batch2d = jnp.pad(batch.astype(jnp.int32).reshape(N, 1),
                      ((0, NP - N), (0, 0)), constant_values=G)
    x_pad = jnp.pad(x, ((0, NP - N), (0, 0)))

    w1xt = conv1_weight[:, :F].T
    w1ct = conv1_weight[:, F:].T
    w2t = conv2_weight.T
    wnt = nnl_weight.T
    wlt = linX_weight.T
    b1 = conv1_bias.reshape(1, F)
    b2 = conv2_bias.reshape(1, F)
    bn = nnl_bias.reshape(1, F)
    bl = linX_bias.reshape(1, 2)
    xcat = x_in[:, 1:9]

    eye = jnp.eye(BLK, dtype=jnp.float32)
    degp = _deg_call(dst_d)
    g1, dinv = _prep(x_pad, degp, eye, batch2d, xcat, w1xt, w1ct)
    p = _scatter_call(g1, src0, dst0, src1, dst1)
    g2 = _comb1(p, g1, dinv, b1, w2t)
    q = _scatter_call(g2, src0, dst0, src1, dst1)
    xs, cnt = _comb2(q, g2, dinv, b2, batch2d)
    return _head(x_in, xs, cnt, wnt, bn, wlt, bl)


# speedup vs baseline: 1.0644x; 1.0644x over previous
import jax
import jax.numpy as jnp
from jax import lax
from jax.experimental import pallas as pl
from jax.experimental.pallas import tpu as pltpu
from jax.experimental.pallas import tpu_sc as plsc

N = 10000
NP = 10240
E = 320000
EP = 327680
G = 64
F = 128
NC, NS = 2, 16
NW = NC * NS
K = 128
NB = 2
EPT = EP // NW
NCH = EPT // K
RPT = NP // NS
SRC_PAD = 10232
DST_PAD = 10239

_MESH = plsc.VectorSubcoreMesh(core_axis_name="c", subcore_axis_name="s")


def _deg_body(dst_hbm, out_hbm, idx_v, hist_v):
    c = lax.axis_index("c")
    s = lax.axis_index("s")
    w = c * NS + s
    pltpu.sync_copy(dst_hbm.at[w], idx_v)

    def zero(i, _):
        hist_v[pl.ds(i * 16, 16)] = jnp.zeros((16,), jnp.float32)
        return 0
    lax.fori_loop(0, NP // 16, zero, 0)

    def add(i, _):
        idx = idx_v[i, :]
        plsc.addupdate_scatter(hist_v, [idx], jnp.ones((16,), jnp.float32))
        return 0
    lax.fori_loop(0, EPT // 16, add, 0)
    pltpu.sync_copy(hist_v, out_hbm.at[w, 0])


_deg_call = pl.kernel(
    _deg_body, mesh=_MESH,
    out_type=jax.ShapeDtypeStruct((NW, 1, NP), jnp.float32),
    scratch_types=[
        pltpu.VMEM((EPT // 16, 16), jnp.int32),
        pltpu.VMEM((NP,), jnp.float32),
    ],
    compiler_params=pltpu.CompilerParams(needs_layout_passes=False),
)


Q0 = 80
Q1 = 80
QMAX = max(Q0, Q1)
R0 = Q0 // NB
R1 = Q1 // NB
RMAX = max(R0, R1)


def _scatter_body(g_hbm, src0_hbm, dst0_hbm, src1_hbm, dst1_hbm, out_hbm,
                  isrc_v, idst_v, rows_v, acc_sh, isem, gsem, ssem):
    c = lax.axis_index("c")
    s = lax.axis_index("s")
    myrounds = jnp.where(c == 0, R0, R1)

    @pl.when(c == 0)
    def _():
        pltpu.sync_copy(src0_hbm.at[s, pl.ds(0, NB)], isrc_v.at[0])
        pltpu.sync_copy(dst0_hbm.at[s, pl.ds(0, NB)], idst_v.at[0])

    @pl.when(c == 1)
    def _():
        pltpu.sync_copy(src1_hbm.at[s, pl.ds(0, NB)], isrc_v.at[0])
        pltpu.sync_copy(dst1_hbm.at[s, pl.ds(0, NB)], idst_v.at[0])

    def zrow(i, _):
        for jj in range(F // 16):
            rows_v[0, i, pl.ds(jj * 16, 16)] = jnp.zeros((16,), jnp.float32)
        return 0
    lax.fori_loop(0, K, zrow, 0, unroll=4)

    def cp(i, _):
        pltpu.sync_copy(rows_v.at[0], acc_sh.at[pl.ds(s * RPT + i * K, K)])
        return 0
    lax.fori_loop(0, RPT // K, cp, 0)
    plsc.subcore_barrier()

    def rnd(r, _):
        slot = lax.rem(r, 2)
        nslot = 1 - slot

        @pl.when(r + 1 < myrounds)
        def _():
            base = (r + 1) * NB

            @pl.when(c == 0)
            def _():
                pltpu.async_copy(src0_hbm.at[s, pl.ds(base, NB)],
                                 isrc_v.at[nslot], isem)
                pltpu.async_copy(dst0_hbm.at[s, pl.ds(base, NB)],
                                 idst_v.at[nslot], isem)

            @pl.when(c == 1)
            def _():
                pltpu.async_copy(src1_hbm.at[s, pl.ds(base, NB)],
                                 isrc_v.at[nslot], isem)
                pltpu.async_copy(dst1_hbm.at[s, pl.ds(base, NB)],
                                 idst_v.at[nslot], isem)

        @pl.when(r < myrounds)
        def _():
            gd = []
            for b in range(NB):
                gd.append(pltpu.async_copy(
                    g_hbm.at[isrc_v.at[slot, b, 0]], rows_v.at[b], gsem))
            sd = []
            for b in range(NB):
                gd[b].wait()
                sd.append(pltpu.async_copy(
                    rows_v.at[b], acc_sh.at[idst_v.at[slot, b, 0]],
                    ssem, add=True))
            for d in sd:
                d.wait()

        @pl.when(r + 1 < myrounds)
        def _():
            pltpu.make_async_copy(src0_hbm.at[s, pl.ds(0, NB)],
                                  isrc_v.at[nslot], isem).wait()
            pltpu.make_async_copy(dst0_hbm.at[s, pl.ds(0, NB)],
                                  idst_v.at[nslot], isem).wait()
        return 0
    lax.fori_loop(0, RMAX, rnd, 0)
    plsc.subcore_barrier()
    pltpu.sync_copy(acc_sh.at[pl.ds(s * RPT, RPT)],
                    out_hbm.at[c, pl.ds(s * RPT, RPT)])


_scatter_call = pl.kernel(
    _scatter_body, mesh=_MESH,
    out_type=jax.ShapeDtypeStruct((NC, NP, F), jnp.float32),
    scratch_types=[
        pltpu.VMEM((2, NB, 1, K), jnp.int32),
        pltpu.VMEM((2, NB, 1, K), jnp.int32),
        pltpu.VMEM((NB, K, F), jnp.float32),
        pltpu.VMEM_SHARED((NP, F), jnp.float32),
        pltpu.SemaphoreType.DMA,
        pltpu.SemaphoreType.DMA,
        pltpu.SemaphoreType.DMA,
    ],
)


BLK = 1024
GRID = NP // BLK


def _prep_body(x_ref, degp_ref, eye_ref, batch_ref, xcat_ref, w1xt_ref,
               w1ct_ref, g1_ref, dinv_ref):
    deg_row = jnp.sum(degp_ref[...], axis=0) + 1.0
    deg = lax.dot_general(eye_ref[...], deg_row, (((1,), (1,)), ((), ())),
                          preferred_element_type=jnp.float32)
    dinv = lax.rsqrt(deg)
    onehot = (batch_ref[...] ==
              lax.broadcasted_iota(jnp.int32, (BLK, G), 1)).astype(jnp.float32)
    xcw = jnp.dot(xcat_ref[...], w1ct_ref[...],
                  preferred_element_type=jnp.float32)
    hw = (jnp.dot(x_ref[...], w1xt_ref[...],
                  preferred_element_type=jnp.float32)
          + jnp.dot(onehot, xcw, preferred_element_type=jnp.float32))
    g1_ref[...] = dinv * hw
    dinv_ref[...] = dinv


def _prep(x, degp, eye, batch2d, xcat, w1xt, w1ct):
    return pl.pallas_call(
        _prep_body,
        grid=(GRID,),
        in_specs=[
            pl.BlockSpec((BLK, F), lambda i: (i, 0)),
            pl.BlockSpec((NW, 1, BLK), lambda i: (0, 0, i)),
            pl.BlockSpec((BLK, BLK), lambda i: (0, 0)),
            pl.BlockSpec((BLK, 1), lambda i: (i, 0)),
            pl.BlockSpec((G, 8), lambda i: (0, 0)),
            pl.BlockSpec((F, F), lambda i: (0, 0)),
            pl.BlockSpec((8, F), lambda i: (0, 0)),
        ],
        out_specs=[
            pl.BlockSpec((BLK, F), lambda i: (i, 0)),
            pl.BlockSpec((BLK, 1), lambda i: (i, 0)),
        ],
        out_shape=[
            jax.ShapeDtypeStruct((NP, F), jnp.float32),
            jax.ShapeDtypeStruct((NP, 1), jnp.float32),
        ],
    )(x, degp, eye, batch2d, xcat, w1xt, w1ct)


def _comb1_body(p_ref, g1_ref, dinv_ref, b1_ref, w2t_ref, g2_ref):
    dinv = dinv_ref[...]
    h1 = jnp.maximum(
        dinv * (p_ref[0] + p_ref[1] + g1_ref[...]) + b1_ref[...], 0.0)
    g2_ref[...] = dinv * jnp.dot(h1, w2t_ref[...],
                                 preferred_element_type=jnp.float32)


def _comb1(p, g1, dinv, b1, w2t):
    return pl.pallas_call(
        _comb1_body,
        grid=(GRID,),
        in_specs=[
            pl.BlockSpec((NC, BLK, F), lambda i: (0, i, 0)),
            pl.BlockSpec((BLK, F), lambda i: (i, 0)),
            pl.BlockSpec((BLK, 1), lambda i: (i, 0)),
            pl.BlockSpec((1, F), lambda i: (0, 0)),
            pl.BlockSpec((F, F), lambda i: (0, 0)),
        ],
        out_specs=pl.BlockSpec((BLK, F), lambda i: (i, 0)),
        out_shape=jax.ShapeDtypeStruct((NP, F), jnp.float32),
    )(p, g1, dinv, b1, w2t)


def _comb2_body(q_ref, g2_ref, dinv_ref, b2_ref, batch_ref, xs_ref, cnt_ref):
    i = pl.program_id(0)
    dinv = dinv_ref[...]
    h2 = jnp.maximum(
        dinv * (q_ref[0] + q_ref[1] + g2_ref[...]) + b2_ref[...], 0.0)
    onehot = (batch_ref[...] ==
              lax.broadcasted_iota(jnp.int32, (BLK, G), 1)).astype(jnp.float32)
    xs_p = lax.dot_general(onehot, h2, (((0,), (0,)), ((), ())),
                           preferred_element_type=jnp.float32)
    cnt_p = jnp.broadcast_to(jnp.sum(onehot, axis=0)[:, None], (G, F))

    @pl.when(i == 0)
    def _():
        xs_ref[...] = xs_p
        cnt_ref[...] = cnt_p

    @pl.when(i > 0)
    def _():
        xs_ref[...] += xs_p
        cnt_ref[...] += cnt_p


def _comb2(q, g2, dinv, b2, batch2d):
    return pl.pallas_call(
        _comb2_body,
        grid=(GRID,),
        in_specs=[
            pl.BlockSpec((NC, BLK, F), lambda i: (0, i, 0)),
            pl.BlockSpec((BLK, F), lambda i: (i, 0)),
            pl.BlockSpec((BLK, 1), lambda i: (i, 0)),
            pl.BlockSpec((1, F), lambda i: (0, 0)),
            pl.BlockSpec((BLK, 1), lambda i: (i, 0)),
        ],
        out_specs=[
            pl.BlockSpec((G, F), lambda i: (0, 0)),
            pl.BlockSpec((G, F), lambda i: (0, 0)),
        ],
        out_shape=[
            jax.ShapeDtypeStruct((G, F), jnp.float32),
            jax.ShapeDtypeStruct((G, F), jnp.float32),
        ],
        compiler_params=pltpu.CompilerParams(
            dimension_semantics=("arbitrary",)),
    )(q, g2, dinv, b2, batch2d)


def _head_body(xin_ref, xs_ref, cnt_ref, wnt_ref, bn_ref, wlt_ref, bl_ref,
               out_ref):
    xs = xs_ref[...] / jnp.maximum(cnt_ref[...], 1.0)
    z = jnp.maximum(jnp.dot(xs, wnt_ref[...],
                            preferred_element_type=jnp.float32)
                    + bn_ref[...], 0.0)
    out = jnp.dot(z, wlt_ref[...], preferred_element_type=jnp.float32) \
        + bl_ref[...]
    dev = xin_ref[:, 0:1]
    out_ref[...] = dev * (1.0 + out[:, 1:2]) - out[:, 0:1]


def _head(x_in, xs, cnt, wnt, bn, wlt, bl):
    return pl.pallas_call(
        _head_body,
        out_shape=jax.ShapeDtypeStruct((G, 1), jnp.float32),
    )(x_in, xs, cnt, wnt, bn, wlt, bl)


def kernel(x_in, x, edge_index, batch, conv1_weight, conv1_bias,
           conv2_weight, conv2_bias, nnl_weight, nnl_bias,
           linX_weight, linX_bias):
    src_flat = jnp.concatenate([
        edge_index[0].astype(jnp.int32),
        jnp.full((EP - E,), SRC_PAD, jnp.int32)])
    dst_flat = jnp.concatenate([
        edge_index[1].astype(jnp.int32),
        jnp.full((EP - E,), DST_PAD, jnp.int32)])
    n0 = NS * Q0 * K
    src0 = src_flat[:n0].reshape(NS, Q0, 1, K)
    dst0 = dst_flat[:n0].reshape(NS, Q0, 1, K)
    src1 = src_flat[n0:].reshape(NS, Q1, 1, K)
    dst1 = dst_flat[n0:].reshape(NS, Q1, 1, K)
    dst_d = dst_flat.reshape(NW, EPT // 16, 16)
    batch2d = jnp.pad(batch.astype(jnp.int32).reshape(N, 1),
                      ((0, NP - N), (0, 0)), constant_values=G)
    x_pad = jnp.pad(x, ((0, NP - N), (0, 0)))

    w1xt = conv1_weight[:, :F].T
    w1ct = conv1_weight[:, F:].T
    w2t = conv2_weight.T
    wnt = nnl_weight.T
    wlt = linX_weight.T
    b1 = conv1_bias.reshape(1, F)
    b2 = conv2_bias.reshape(1, F)
    bn = nnl_bias.reshape(1, F)
    bl = linX_bias.reshape(1, 2)
    xcat = x_in[:, 1:9]

    eye = jnp.eye(BLK, dtype=jnp.float32)
    degp = _deg_call(dst_d)
    g1, dinv = _prep(x_pad, degp, eye, batch2d, xcat, w1xt, w1ct)
    p = _scatter_call(g1, src0, dst0, src1, dst1)
    g2 = _comb1(p, g1, dinv, b1, w2t)
    q = _scatter_call(g2, src0, dst0, src1, dst1)
    xs, cnt = _comb2(q, g2, dinv, b2, batch2d)
    return _head(x_in, xs, cnt, wnt, bn, wlt, bl)
